# baseline (speedup 1.0000x reference)
import functools

import jax
import jax.numpy as jnp
import numpy as np
from jax.experimental import pallas as pl
from jax.experimental.pallas import tpu as pltpu
from jax.experimental.shard_map import shard_map
from jax.sharding import Mesh, PartitionSpec as P

_M_TILE = 1024
_H_TILE = 512


_SUB = 2


def _mlp_body(x_ref, w1_ref, b1_ref, w2_ref, b2_ref, o_ref):
    h_idx = pl.program_id(1)
    x = x_ref[...]
    sub = w1_ref.shape[0] // _SUB

    def chain(k):
        lo = k * sub
        t = jax.lax.dot_general(
            x, w1_ref[lo:lo + sub, :].astype(jnp.bfloat16),
            dimension_numbers=(((1,), (1,)), ((), ())),
            preferred_element_type=jnp.float32,
        )
        t = t + b1_ref[:, lo:lo + sub]
        t = 0.5 * t * (1.0 + jax.lax.erf(t * 0.7071067811865476))
        return jax.lax.dot_general(
            t.astype(jnp.bfloat16), w2_ref[:, lo:lo + sub].astype(jnp.bfloat16),
            dimension_numbers=(((1,), (1,)), ((), ())),
            preferred_element_type=jnp.float32,
        )

    contrib = chain(0)
    for k in range(1, _SUB):
        contrib = contrib + chain(k)

    @pl.when(h_idx == 0)
    def _init():
        o_ref[...] = contrib + b2_ref[...]

    @pl.when(h_idx != 0)
    def _accum():
        o_ref[...] += contrib


@functools.partial(jax.jit, static_argnames=("m_tile", "h_tile"))
def _fused_mlp(xm, w1, b1, w2, b2, m_tile=_M_TILE, h_tile=_H_TILE):
    m, embed = xm.shape
    hidden = w1.shape[0]
    grid = (m // m_tile, hidden // h_tile)
    return pl.pallas_call(
        _mlp_body,
        grid=grid,
        in_specs=[
            pl.BlockSpec((m_tile, embed), lambda i, j: (i, 0)),
            pl.BlockSpec((h_tile, embed), lambda i, j: (j, 0)),
            pl.BlockSpec((1, h_tile), lambda i, j: (0, j)),
            pl.BlockSpec((embed, h_tile), lambda i, j: (0, j)),
            pl.BlockSpec((1, embed), lambda i, j: (0, 0)),
        ],
        out_specs=pl.BlockSpec((m_tile, embed), lambda i, j: (i, 0)),
        out_shape=jax.ShapeDtypeStruct((m, embed), jnp.float32),
        compiler_params=pltpu.CompilerParams(
            dimension_semantics=("parallel", "arbitrary"),
        ),
    )(xm, w1, b1, w2, b2)


def _shard_fn(x, proj1, proj1_bias, proj2, proj2_bias):
    length, n, embed = x.shape
    xm = x.reshape(length * n, embed).astype(jnp.bfloat16)
    out = _fused_mlp(xm, proj1, proj1_bias.reshape(1, -1),
                     proj2, proj2_bias.reshape(1, -1))
    return out.reshape(length, n, embed)


def kernel(x, proj1, proj1_bias, proj2, proj2_bias, gate_w):
    del gate_w
    length = x.shape[0]
    w1 = proj1.astype(jnp.bfloat16)
    w2 = proj2.astype(jnp.bfloat16)
    devs = jax.devices()
    n_shards = 2 if len(devs) >= 2 and length % 2 == 0 else 1
    mesh = Mesh(np.array(devs[:n_shards]), ("d",))
    sharded = shard_map(
        _shard_fn,
        mesh=mesh,
        in_specs=(P("d", None, None), P(None, None), P(None),
                  P(None, None), P(None)),
        out_specs=P("d", None, None),
        check_rep=False,
    )
    return sharded(x, w1, proj1_bias, w2, proj2_bias)


# speedup vs baseline: 1.5038x; 1.5038x over previous
import functools

import jax
import jax.numpy as jnp
from jax.experimental import pallas as pl
from jax.experimental.pallas import tpu as pltpu

_M_TILE = 1024
_H_TILE = 512


_SUB = 2


def _mlp_body(x_ref, w1_ref, b1_ref, w2_ref, b2_ref, o_ref, c_ref):
    h_idx = pl.program_id(1)
    x = x_ref[...]
    sub = w1_ref.shape[0] // _SUB

    def chain(k):
        lo = k * sub
        t = jax.lax.dot_general(
            x, w1_ref[lo:lo + sub, :].astype(jnp.bfloat16),
            dimension_numbers=(((1,), (1,)), ((), ())),
            preferred_element_type=jnp.float32,
        )
        t = t + b1_ref[:, lo:lo + sub]
        t = 0.5 * t * (1.0 + jax.lax.erf(t * 0.7071067811865476))
        return jax.lax.dot_general(
            t.astype(jnp.bfloat16), w2_ref[:, lo:lo + sub].astype(jnp.bfloat16),
            dimension_numbers=(((1,), (1,)), ((), ())),
            preferred_element_type=jnp.float32,
        )

    contrib = chain(0)
    for k in range(1, _SUB):
        contrib = contrib + chain(k)

    n_h = pl.num_programs(1)
    slot = jax.lax.rem(h_idx, 2)
    prev = jax.lax.rem(h_idx + 1, 2)

    @pl.when(h_idx == 0)
    def _init():
        o_ref[...] = jnp.broadcast_to(b2_ref[...], o_ref.shape)

    @pl.when(h_idx > 0)
    def _fold_prev():
        o_ref[...] += c_ref[prev].astype(jnp.float32)

    @pl.when(h_idx < n_h - 1)
    def _stage():
        c_ref[slot] = contrib.astype(jnp.bfloat16)

    @pl.when(h_idx == n_h - 1)
    def _fold_last():
        o_ref[...] += contrib


@functools.partial(jax.jit, static_argnames=("m_tile", "h_tile"))
def _fused_mlp(xm, w1, b1, w2, b2, m_tile=_M_TILE, h_tile=_H_TILE):
    m, embed = xm.shape
    hidden = w1.shape[0]
    grid = (m // m_tile, hidden // h_tile)
    return pl.pallas_call(
        _mlp_body,
        grid=grid,
        in_specs=[
            pl.BlockSpec((m_tile, embed), lambda i, j: (i, 0)),
            pl.BlockSpec((h_tile, embed), lambda i, j: (j, 0)),
            pl.BlockSpec((1, h_tile), lambda i, j: (0, j)),
            pl.BlockSpec((embed, h_tile), lambda i, j: (0, j)),
            pl.BlockSpec((1, embed), lambda i, j: (0, 0)),
        ],
        out_specs=pl.BlockSpec((m_tile, embed), lambda i, j: (i, 0)),
        out_shape=jax.ShapeDtypeStruct((m, embed), jnp.float32),
        scratch_shapes=[pltpu.VMEM((2, m_tile, embed), jnp.bfloat16)],
        compiler_params=pltpu.CompilerParams(
            dimension_semantics=("parallel", "arbitrary"),
        ),
    )(xm, w1, b1, w2, b2)


def kernel(x, proj1, proj1_bias, proj2, proj2_bias, gate_w):
    del gate_w
    length, n, embed = x.shape
    xm = x.reshape(length * n, embed).astype(jnp.bfloat16)
    out = _fused_mlp(
        xm,
        proj1,
        proj1_bias.reshape(1, -1),
        proj2,
        proj2_bias.reshape(1, -1),
    )
    return out.reshape(length, n, embed)


# speedup vs baseline: 1.5698x; 1.0439x over previous
import functools

import jax
import jax.numpy as jnp
from jax.experimental import pallas as pl
from jax.experimental.pallas import tpu as pltpu

_M_TILE = 1024
_H_TILE = 512


_SUB = 2


def _mlp_body(x_ref, w1_ref, b1_ref, w2_ref, b2_ref, o_ref):
    h_idx = pl.program_id(1)
    x = x_ref[...]
    sub = w1_ref.shape[0] // _SUB

    def chain(k):
        lo = k * sub
        t = jax.lax.dot_general(
            x, w1_ref[lo:lo + sub, :].astype(jnp.bfloat16),
            dimension_numbers=(((1,), (1,)), ((), ())),
            preferred_element_type=jnp.float32,
        )
        t = t + b1_ref[:, lo:lo + sub]
        t = 0.5 * t * (1.0 + jax.lax.erf(t * 0.7071067811865476))
        return jax.lax.dot_general(
            t.astype(jnp.bfloat16), w2_ref[:, lo:lo + sub].astype(jnp.bfloat16),
            dimension_numbers=(((1,), (1,)), ((), ())),
            preferred_element_type=jnp.float32,
        )

    contrib = chain(0)
    for k in range(1, _SUB):
        contrib = contrib + chain(k)

    @pl.when(h_idx == 0)
    def _init():
        o_ref[...] = contrib + b2_ref[...]

    @pl.when(h_idx != 0)
    def _accum():
        o_ref[...] += contrib


@functools.partial(jax.jit, static_argnames=("m_tile", "h_tile"))
def _fused_mlp(xm, w1, b1, w2, b2, m_tile=_M_TILE, h_tile=_H_TILE):
    m, embed = xm.shape
    hidden = w1.shape[0]
    grid = (m // m_tile, hidden // h_tile)
    return pl.pallas_call(
        _mlp_body,
        grid=grid,
        in_specs=[
            pl.BlockSpec((m_tile, embed), lambda i, j: (i, 0)),
            pl.BlockSpec((h_tile, embed), lambda i, j: (j, 0)),
            pl.BlockSpec((1, h_tile), lambda i, j: (0, j)),
            pl.BlockSpec((embed, h_tile), lambda i, j: (0, j)),
            pl.BlockSpec((1, embed), lambda i, j: (0, 0)),
        ],
        out_specs=pl.BlockSpec((m_tile, embed), lambda i, j: (i, 0)),
        out_shape=jax.ShapeDtypeStruct((m, embed), jnp.float32),
        compiler_params=pltpu.CompilerParams(
            dimension_semantics=("parallel", "arbitrary"),
        ),
    )(xm, w1, b1, w2, b2)


def kernel(x, proj1, proj1_bias, proj2, proj2_bias, gate_w):
    del gate_w
    length, n, embed = x.shape
    xm = x.reshape(length * n, embed).astype(jnp.bfloat16)
    out = _fused_mlp(
        xm,
        proj1,
        proj1_bias.reshape(1, -1),
        proj2,
        proj2_bias.reshape(1, -1),
    )
    return out.reshape(length, n, embed)
